# baseline (speedup 1.0000x reference)
import functools

import jax
import jax.numpy as jnp
from jax import lax
from jax.experimental import pallas as pl
from jax.experimental.pallas import tpu as pltpu
from jax.experimental.pallas import tpu_sc as plsc

_N = 10000
_H = 128
_C = 40
_L = 3
_E = 320000

_NC = 2
_NS = 16
_CH = 80
_NCH = 128
_EPW = _NCH * _CH
_EPAD = _NC * _NS * _EPW
_NPAD = 10240
_RPT = _NPAD // _NS


def _seg_sum_partials(h, src, dst):
    mesh = plsc.VectorSubcoreMesh(core_axis_name="c", subcore_axis_name="s")

    @functools.partial(
        pl.kernel,
        mesh=mesh,
        out_type=jax.ShapeDtypeStruct((_NC, _NPAD, _H), jnp.float32),
        scratch_types=[
            pltpu.VMEM((_EPW,), jnp.int32),
            pltpu.VMEM((_NCH, _CH), jnp.int32),
            pltpu.VMEM((_CH, _H), jnp.float32),
            pltpu.VMEM((_CH, _H), jnp.float32),
            pltpu.VMEM_SHARED((_NPAD, _H), jnp.float32),
            pltpu.SemaphoreType.DMA,
            pltpu.SemaphoreType.DMA,
        ],
    )
    def seg(h_hbm, src_hbm, dst_hbm, out_hbm, src_v, dst_v, rows0, rows1, acc,
            sem0, sem1):
        c = lax.axis_index("c")
        s = lax.axis_index("s")

        def zf(i, carry):
            rows0[i // 8, pl.ds((i % 8) * 16, 16)] = jnp.zeros((16,), jnp.float32)
            return carry

        lax.fori_loop(0, _CH * 8, zf, 0)

        row0 = s * _RPT
        for k in range(_RPT // _CH):
            pltpu.sync_copy(rows0, acc.at[pl.ds(row0 + k * _CH, _CH)])

        pltpu.sync_copy(src_hbm.at[c, s], src_v)
        pltpu.sync_copy(dst_hbm.at[c, s], dst_v)
        plsc.subcore_barrier()

        def sidx(j):
            return src_v.at[pl.ds(j * _CH, _CH)]

        pltpu.async_copy(h_hbm.at[sidx(0)], rows0, sem0)
        pltpu.async_copy(h_hbm.at[sidx(1)], rows1, sem1)

        def body(k, carry):
            j = 2 * k
            pltpu.make_async_copy(h_hbm.at[sidx(j)], rows0, sem0).wait()
            pltpu.sync_copy(rows0, acc.at[dst_v.at[j]], add=True)
            pltpu.async_copy(h_hbm.at[sidx(j + 2)], rows0, sem0)
            pltpu.make_async_copy(h_hbm.at[sidx(j + 1)], rows1, sem1).wait()
            pltpu.sync_copy(rows1, acc.at[dst_v.at[j + 1]], add=True)
            pltpu.async_copy(h_hbm.at[sidx(j + 3)], rows1, sem1)
            return carry

        lax.fori_loop(0, (_NCH - 2) // 2, body, 0)
        pltpu.make_async_copy(h_hbm.at[sidx(_NCH - 2)], rows0, sem0).wait()
        pltpu.sync_copy(rows0, acc.at[dst_v.at[_NCH - 2]], add=True)
        pltpu.make_async_copy(h_hbm.at[sidx(_NCH - 1)], rows1, sem1).wait()
        pltpu.sync_copy(rows1, acc.at[dst_v.at[_NCH - 1]], add=True)

        plsc.subcore_barrier()
        pltpu.sync_copy(acc.at[pl.ds(row0, _RPT)], out_hbm.at[c, pl.ds(row0, _RPT)])

    return seg(h, src, dst)


def _mlp_body(h_ref, p_ref, eps_ref, w1_ref, b1_ref, g1_ref, be1_ref,
              w2_ref, b2_ref, g2_ref, be2_ref, o_ref):
    u = (1.0 + eps_ref[0, 0]) * h_ref[...] + p_ref[0, :_N] + p_ref[1, :_N]
    h1 = jnp.dot(u, w1_ref[...], preferred_element_type=jnp.float32) + b1_ref[...]
    h1 = jnp.maximum(h1, 0.0)
    mu = jnp.mean(h1, axis=0, keepdims=True)
    var = jnp.mean((h1 - mu) ** 2, axis=0, keepdims=True)
    h1 = g1_ref[...] * (h1 - mu) * lax.rsqrt(var + 1e-5) + be1_ref[...]
    h2 = jnp.dot(h1, w2_ref[...], preferred_element_type=jnp.float32) + b2_ref[...]
    h2 = jnp.maximum(h2, 0.0)
    mu2 = jnp.mean(h2, axis=0, keepdims=True)
    var2 = jnp.mean((h2 - mu2) ** 2, axis=0, keepdims=True)
    o_ref[...] = g2_ref[...] * (h2 - mu2) * lax.rsqrt(var2 + 1e-5) + be2_ref[...]


def _mlp_tc(h, parts, eps, w1, b1, g1, be1, w2, b2, g2, be2):
    return pl.pallas_call(
        _mlp_body,
        out_shape=jax.ShapeDtypeStruct((_N, _H), jnp.float32),
    )(h, parts, eps.reshape(1, 1), w1, b1.reshape(1, _H), g1.reshape(1, _H),
      be1.reshape(1, _H), w2, b2.reshape(1, _H), g2.reshape(1, _H),
      be2.reshape(1, _H))


def _head_body(h_ref, w1_ref, b1_ref, w2_ref, b2_ref, o_ref):
    z = jnp.dot(h_ref[...], w1_ref[...], preferred_element_type=jnp.float32)
    z = jnp.maximum(z + b1_ref[...], 0.0)
    z = jnp.dot(z, w2_ref[...], preferred_element_type=jnp.float32) + b2_ref[...]
    m = jnp.max(z, axis=-1, keepdims=True)
    e = z - m
    o_ref[...] = e - jnp.log(jnp.sum(jnp.exp(e), axis=-1, keepdims=True))


def _head_tc(h, w1, b1, w2, b2):
    return pl.pallas_call(
        _head_body,
        out_shape=jax.ShapeDtypeStruct((_N, _C), jnp.float32),
    )(h, w1, b1.reshape(1, _H), w2, b2.reshape(1, _C))


def kernel(x, edge_index, params):
    src = jnp.pad(edge_index[0].astype(jnp.int32), (0, _EPAD - _E))
    dst = jnp.pad(edge_index[1].astype(jnp.int32), (0, _EPAD - _E),
                  constant_values=_NPAD - 1)
    src = src.reshape(_NC, _NS, _EPW)
    dst = dst.reshape(_NC, _NS, _NCH, _CH)
    h = x
    for l in range(_L):
        pre = "conv%d" % l
        parts = _seg_sum_partials(h, src, dst)
        h = _mlp_tc(
            h, parts, params[pre + "_eps"],
            params[pre + "_W1"], params[pre + "_b1"],
            params[pre + "_g1"], params[pre + "_be1"],
            params[pre + "_W2"], params[pre + "_b2"],
            params[pre + "_g2"], params[pre + "_be2"],
        )
    return _head_tc(h, params["lin1_W"], params["lin1_b"],
                    params["lin2_W"], params["lin2_b"])


# speedup vs baseline: 3.6337x; 3.6337x over previous
import functools

import jax
import jax.numpy as jnp
from jax import lax
from jax.experimental import pallas as pl
from jax.experimental.pallas import tpu as pltpu
from jax.experimental.pallas import tpu_sc as plsc

_N = 10000
_H = 128
_C = 40
_L = 3
_E = 320000

_NC = 2
_NS = 16
_CH = 80
_NCH = 128
_EPW = _NCH * _CH
_EPAD = _NC * _NS * _EPW
_NPAD = 10240
_RPT = _NPAD // _NS


def _seg_sum_partials(h, src, dst):
    mesh = plsc.VectorSubcoreMesh(core_axis_name="c", subcore_axis_name="s")

    @functools.partial(
        pl.kernel,
        mesh=mesh,
        out_type=jax.ShapeDtypeStruct((_NC, _NPAD, _H), jnp.float32),
        scratch_types=[
            pltpu.VMEM((_EPW,), jnp.int32),
            pltpu.VMEM((_NCH, _CH), jnp.int32),
            pltpu.VMEM((_CH, _H), jnp.float32),
            pltpu.VMEM((_CH, _H), jnp.float32),
            pltpu.VMEM_SHARED((_NPAD, _H), jnp.float32),
            pltpu.SemaphoreType.DMA,
            pltpu.SemaphoreType.DMA,
        ],
    )
    def seg(h_hbm, src_hbm, dst_hbm, out_hbm, src_v, dst_v, rows0, rows1, acc,
            sem0, sem1):
        c = lax.axis_index("c")
        s = lax.axis_index("s")

        def zf(i, carry):
            rows0[i // 8, pl.ds((i % 8) * 16, 16)] = jnp.zeros((16,), jnp.float32)
            return carry

        lax.fori_loop(0, _CH * 8, zf, 0)

        row0 = s * _RPT
        for k in range(_RPT // _CH):
            pltpu.sync_copy(rows0, acc.at[pl.ds(row0 + k * _CH, _CH)])

        pltpu.sync_copy(src_hbm.at[c, s], src_v)
        pltpu.sync_copy(dst_hbm.at[c, s], dst_v)
        plsc.subcore_barrier()

        def sidx(j):
            return src_v.at[pl.ds(j * _CH, _CH)]

        pltpu.async_copy(h_hbm.at[sidx(0)], rows0, sem0)
        pltpu.async_copy(h_hbm.at[sidx(1)], rows1, sem1)

        def body(k, carry):
            j = 2 * k
            pltpu.make_async_copy(h_hbm.at[sidx(j)], rows0, sem0).wait()
            pltpu.sync_copy(rows0, acc.at[dst_v.at[j]], add=True)
            pltpu.async_copy(h_hbm.at[sidx(j + 2)], rows0, sem0)
            pltpu.make_async_copy(h_hbm.at[sidx(j + 1)], rows1, sem1).wait()
            pltpu.sync_copy(rows1, acc.at[dst_v.at[j + 1]], add=True)
            pltpu.async_copy(h_hbm.at[sidx(j + 3)], rows1, sem1)
            return carry

        lax.fori_loop(0, (_NCH - 2) // 2, body, 0)
        pltpu.make_async_copy(h_hbm.at[sidx(_NCH - 2)], rows0, sem0).wait()
        pltpu.sync_copy(rows0, acc.at[dst_v.at[_NCH - 2]], add=True)
        pltpu.make_async_copy(h_hbm.at[sidx(_NCH - 1)], rows1, sem1).wait()
        pltpu.sync_copy(rows1, acc.at[dst_v.at[_NCH - 1]], add=True)

        plsc.subcore_barrier()
        pltpu.sync_copy(acc.at[pl.ds(row0, _RPT)], out_hbm.at[c, pl.ds(row0, _RPT)])

    return seg(h, src, dst)


def _mlp_body(h_ref, p_ref, eps_ref, w1_ref, b1_ref, g1_ref, be1_ref,
              w2_ref, b2_ref, g2_ref, be2_ref, o_ref):
    u = (1.0 + eps_ref[0, 0]) * h_ref[...] + p_ref[0, :_N] + p_ref[1, :_N]
    h1 = jnp.dot(u, w1_ref[...], preferred_element_type=jnp.float32) + b1_ref[...]
    h1 = jnp.maximum(h1, 0.0)
    mu = jnp.mean(h1, axis=0, keepdims=True)
    var = jnp.mean((h1 - mu) ** 2, axis=0, keepdims=True)
    h1 = g1_ref[...] * (h1 - mu) * lax.rsqrt(var + 1e-5) + be1_ref[...]
    h2 = jnp.dot(h1, w2_ref[...], preferred_element_type=jnp.float32) + b2_ref[...]
    h2 = jnp.maximum(h2, 0.0)
    mu2 = jnp.mean(h2, axis=0, keepdims=True)
    var2 = jnp.mean((h2 - mu2) ** 2, axis=0, keepdims=True)
    o_ref[...] = g2_ref[...] * (h2 - mu2) * lax.rsqrt(var2 + 1e-5) + be2_ref[...]


def _mlp_tc(h, parts, eps, w1, b1, g1, be1, w2, b2, g2, be2):
    return pl.pallas_call(
        _mlp_body,
        out_shape=jax.ShapeDtypeStruct((_N, _H), jnp.float32),
    )(h, parts, eps.reshape(1, 1), w1, b1.reshape(1, _H), g1.reshape(1, _H),
      be1.reshape(1, _H), w2, b2.reshape(1, _H), g2.reshape(1, _H),
      be2.reshape(1, _H))


def _head_body(h_ref, w1_ref, b1_ref, w2_ref, b2_ref, o_ref):
    z = jnp.dot(h_ref[...], w1_ref[...], preferred_element_type=jnp.float32)
    z = jnp.maximum(z + b1_ref[...], 0.0)
    z = jnp.dot(z, w2_ref[...], preferred_element_type=jnp.float32) + b2_ref[...]
    m = jnp.max(z, axis=-1, keepdims=True)
    e = z - m
    o_ref[...] = e - jnp.log(jnp.sum(jnp.exp(e), axis=-1, keepdims=True))


def _head_tc(h, w1, b1, w2, b2):
    return pl.pallas_call(
        _head_body,
        out_shape=jax.ShapeDtypeStruct((_N, _C), jnp.float32),
    )(h, w1, b1.reshape(1, _H), w2, b2.reshape(1, _C))


def kernel(x, edge_index, params):
    pad_i = jnp.arange(_EPAD - _E, dtype=jnp.int32)
    src = jnp.concatenate([edge_index[0].astype(jnp.int32), pad_i % _N])
    dst = jnp.concatenate([edge_index[1].astype(jnp.int32),
                           _N + pad_i % (_NPAD - _N)])
    src = src.reshape(_NC, _NS, _EPW)
    dst = dst.reshape(_NC, _NS, _NCH, _CH)
    h = x
    for l in range(_L):
        pre = "conv%d" % l
        parts = _seg_sum_partials(h, src, dst)
        h = _mlp_tc(
            h, parts, params[pre + "_eps"],
            params[pre + "_W1"], params[pre + "_b1"],
            params[pre + "_g1"], params[pre + "_be1"],
            params[pre + "_W2"], params[pre + "_b2"],
            params[pre + "_g2"], params[pre + "_be2"],
        )
    return _head_tc(h, params["lin1_W"], params["lin1_b"],
                    params["lin2_W"], params["lin2_b"])


# speedup vs baseline: 3.7485x; 1.0316x over previous
import functools

import jax
import jax.numpy as jnp
from jax import lax
from jax.experimental import pallas as pl
from jax.experimental.pallas import tpu as pltpu
from jax.experimental.pallas import tpu_sc as plsc

_N = 10000
_H = 128
_C = 40
_L = 3
_E = 320000

_NC = 2
_NS = 16
_CH = 80
_G = 32
_NG = 4
_NCH = _NG * _G
_EPW = _NCH * _CH
_EPAD = _NC * _NS * _EPW
_NPAD = 10240
_RPT = _NPAD // _NS


def _seg_sum_partials(h, src, dst):
    mesh = plsc.VectorSubcoreMesh(core_axis_name="c", subcore_axis_name="s")

    @functools.partial(
        pl.kernel,
        mesh=mesh,
        out_type=jax.ShapeDtypeStruct((_NC, _NPAD, _H), jnp.float32),
        scratch_types=[
            pltpu.VMEM((_G * _CH,), jnp.int32),
            pltpu.VMEM((_G, _CH), jnp.int32),
            pltpu.VMEM((_CH, _H), jnp.float32),
            pltpu.VMEM((_CH, _H), jnp.float32),
            pltpu.VMEM((_CH, _H), jnp.float32),
            pltpu.VMEM((_CH, _H), jnp.float32),
            pltpu.VMEM_SHARED((_NPAD, _H), jnp.float32),
            pltpu.SemaphoreType.DMA,
            pltpu.SemaphoreType.DMA,
            pltpu.SemaphoreType.DMA,
            pltpu.SemaphoreType.DMA,
            pltpu.SemaphoreType.DMA,
            pltpu.SemaphoreType.DMA,
            pltpu.SemaphoreType.DMA,
            pltpu.SemaphoreType.DMA,
        ],
    )
    def seg(h_hbm, src_hbm, dst_hbm, out_hbm, src_v, dst_v, b0, b1, b2, b3,
            acc, g0, g1, g2, g3, s0, s1, s2, s3):
        c = lax.axis_index("c")
        s = lax.axis_index("s")
        bufs = (b0, b1, b2, b3)
        gsem = (g0, g1, g2, g3)
        ssem = (s0, s1, s2, s3)

        def zf(i, carry):
            b0[i // 8, pl.ds((i % 8) * 16, 16)] = jnp.zeros((16,), jnp.float32)
            return carry

        lax.fori_loop(0, _CH * 8, zf, 0)

        row0 = s * _RPT
        for k in range(_RPT // _CH):
            pltpu.sync_copy(b0, acc.at[pl.ds(row0 + k * _CH, _CH)])

        plsc.subcore_barrier()

        def sidx(j):
            return src_v.at[pl.ds(j * _CH, _CH)]

        def gather(j, i):
            pltpu.async_copy(h_hbm.at[sidx(j)], bufs[i], gsem[i])

        def gwait(j, i):
            pltpu.make_async_copy(h_hbm.at[sidx(j)], bufs[i], gsem[i]).wait()

        def scat(j, i):
            pltpu.async_copy(bufs[i], acc.at[dst_v.at[j]], ssem[i], add=True)

        def swait(j, i):
            pltpu.make_async_copy(bufs[i], acc.at[dst_v.at[j]], ssem[i]).wait()

        for g in range(_NG):
            pltpu.sync_copy(src_hbm.at[c, s, g], src_v)
            pltpu.sync_copy(dst_hbm.at[c, s, g], dst_v)

            for i in range(4):
                gather(i, i)

            def body(k, carry):
                j = 4 * k
                gwait(j, 0)
                scat(j, 0)
                gwait(j + 1, 1)
                scat(j + 1, 1)
                swait(j, 0)
                gather(j + 4, 0)
                swait(j + 1, 1)
                gather(j + 5, 1)
                gwait(j + 2, 2)
                scat(j + 2, 2)
                gwait(j + 3, 3)
                scat(j + 3, 3)
                swait(j + 2, 2)
                gather(j + 6, 2)
                swait(j + 3, 3)
                gather(j + 7, 3)
                return carry

            lax.fori_loop(0, _G // 4 - 1, body, 0)
            base = _G - 4
            for i in range(4):
                gwait(base + i, i)
                scat(base + i, i)
            for i in range(4):
                swait(base + i, i)

        plsc.subcore_barrier()
        pltpu.sync_copy(acc.at[pl.ds(row0, _RPT)], out_hbm.at[c, pl.ds(row0, _RPT)])

    return seg(h, src, dst)


def _mlp_body(h_ref, p_ref, eps_ref, w1_ref, b1_ref, g1_ref, be1_ref,
              w2_ref, b2_ref, g2_ref, be2_ref, o_ref):
    u = (1.0 + eps_ref[0, 0]) * h_ref[...] + p_ref[0, :_N] + p_ref[1, :_N]
    h1 = jnp.dot(u, w1_ref[...], preferred_element_type=jnp.float32) + b1_ref[...]
    h1 = jnp.maximum(h1, 0.0)
    mu = jnp.mean(h1, axis=0, keepdims=True)
    var = jnp.mean((h1 - mu) ** 2, axis=0, keepdims=True)
    h1 = g1_ref[...] * (h1 - mu) * lax.rsqrt(var + 1e-5) + be1_ref[...]
    h2 = jnp.dot(h1, w2_ref[...], preferred_element_type=jnp.float32) + b2_ref[...]
    h2 = jnp.maximum(h2, 0.0)
    mu2 = jnp.mean(h2, axis=0, keepdims=True)
    var2 = jnp.mean((h2 - mu2) ** 2, axis=0, keepdims=True)
    o_ref[...] = g2_ref[...] * (h2 - mu2) * lax.rsqrt(var2 + 1e-5) + be2_ref[...]


def _mlp_tc(h, parts, eps, w1, b1, g1, be1, w2, b2, g2, be2):
    return pl.pallas_call(
        _mlp_body,
        out_shape=jax.ShapeDtypeStruct((_N, _H), jnp.float32),
    )(h, parts, eps.reshape(1, 1), w1, b1.reshape(1, _H), g1.reshape(1, _H),
      be1.reshape(1, _H), w2, b2.reshape(1, _H), g2.reshape(1, _H),
      be2.reshape(1, _H))


def _head_body(h_ref, w1_ref, b1_ref, w2_ref, b2_ref, o_ref):
    z = jnp.dot(h_ref[...], w1_ref[...], preferred_element_type=jnp.float32)
    z = jnp.maximum(z + b1_ref[...], 0.0)
    z = jnp.dot(z, w2_ref[...], preferred_element_type=jnp.float32) + b2_ref[...]
    m = jnp.max(z, axis=-1, keepdims=True)
    e = z - m
    o_ref[...] = e - jnp.log(jnp.sum(jnp.exp(e), axis=-1, keepdims=True))


def _head_tc(h, w1, b1, w2, b2):
    return pl.pallas_call(
        _head_body,
        out_shape=jax.ShapeDtypeStruct((_N, _C), jnp.float32),
    )(h, w1, b1.reshape(1, _H), w2, b2.reshape(1, _C))


def kernel(x, edge_index, params):
    pad_i = jnp.arange(_EPAD - _E, dtype=jnp.int32)
    src = jnp.concatenate([edge_index[0].astype(jnp.int32), pad_i % _N])
    dst = jnp.concatenate([edge_index[1].astype(jnp.int32),
                           _N + pad_i % (_NPAD - _N)])
    src = src.reshape(_NC, _NS, _NG, _G * _CH)
    dst = dst.reshape(_NC, _NS, _NG, _G, _CH)
    h = x
    for l in range(_L):
        pre = "conv%d" % l
        parts = _seg_sum_partials(h, src, dst)
        h = _mlp_tc(
            h, parts, params[pre + "_eps"],
            params[pre + "_W1"], params[pre + "_b1"],
            params[pre + "_g1"], params[pre + "_be1"],
            params[pre + "_W2"], params[pre + "_b2"],
            params[pre + "_g2"], params[pre + "_be2"],
        )
    return _head_tc(h, params["lin1_W"], params["lin1_b"],
                    params["lin2_W"], params["lin2_b"])


# speedup vs baseline: 3.8442x; 1.0255x over previous
import functools

import jax
import jax.numpy as jnp
from jax import lax
from jax.experimental import pallas as pl
from jax.experimental.pallas import tpu as pltpu
from jax.experimental.pallas import tpu_sc as plsc

_N = 10000
_H = 128
_C = 40
_L = 3
_E = 320000

_NC = 2
_NS = 16
_CH = 128
_G = 40
_NG = 2
_NCH = _NG * _G
_EPW = _NCH * _CH
_EPAD = _NC * _NS * _EPW
_NPAD = 10240
_RPT = _NPAD // _NS


def _seg_sum_partials(h, src, dst):
    mesh = plsc.VectorSubcoreMesh(core_axis_name="c", subcore_axis_name="s")

    @functools.partial(
        pl.kernel,
        mesh=mesh,
        out_type=jax.ShapeDtypeStruct((_NC, _NPAD, _H), jnp.float32),
        scratch_types=[
            pltpu.VMEM((_G * _CH,), jnp.int32),
            pltpu.VMEM((_G, _CH), jnp.int32),
            pltpu.VMEM((_CH, _H), jnp.float32),
            pltpu.VMEM((_CH, _H), jnp.float32),
            pltpu.VMEM_SHARED((_NPAD, _H), jnp.float32),
            pltpu.SemaphoreType.DMA,
            pltpu.SemaphoreType.DMA,
        ],
    )
    def seg(h_hbm, src_hbm, dst_hbm, out_hbm, src_v, dst_v, b0, b1,
            acc, g0, g1):
        c = lax.axis_index("c")
        s = lax.axis_index("s")
        bufs = (b0, b1)
        gsem = (g0, g1)

        def zf(i, carry):
            b0[i // 8, pl.ds((i % 8) * 16, 16)] = jnp.zeros((16,), jnp.float32)
            return carry

        lax.fori_loop(0, _CH * 8, zf, 0)

        row0 = s * _RPT
        for k in range(_RPT // _CH):
            pltpu.sync_copy(b0, acc.at[pl.ds(row0 + k * _CH, _CH)])

        plsc.subcore_barrier()

        def sidx(j):
            return src_v.at[pl.ds(j * _CH, _CH)]

        def gather(j, i):
            pltpu.async_copy(h_hbm.at[sidx(j)], bufs[i], gsem[i])

        def gwait(j, i):
            pltpu.make_async_copy(h_hbm.at[sidx(j)], bufs[i], gsem[i]).wait()

        def scat(j, i):
            pltpu.sync_copy(bufs[i], acc.at[dst_v.at[j]], add=True)

        for g in range(_NG):
            pltpu.sync_copy(src_hbm.at[c, s, g], src_v)
            pltpu.sync_copy(dst_hbm.at[c, s, g], dst_v)

            gather(0, 0)
            gather(1, 1)

            def body(k, carry):
                j = 2 * k
                gwait(j, 0)
                scat(j, 0)
                gather(j + 2, 0)
                gwait(j + 1, 1)
                scat(j + 1, 1)
                gather(j + 3, 1)
                return carry

            lax.fori_loop(0, _G // 2 - 1, body, 0)
            gwait(_G - 2, 0)
            scat(_G - 2, 0)
            gwait(_G - 1, 1)
            scat(_G - 1, 1)

        plsc.subcore_barrier()
        pltpu.sync_copy(acc.at[pl.ds(row0, _RPT)], out_hbm.at[c, pl.ds(row0, _RPT)])

    return seg(h, src, dst)


def _mlp_body(h_ref, p_ref, eps_ref, w1_ref, b1_ref, g1_ref, be1_ref,
              w2_ref, b2_ref, g2_ref, be2_ref, o_ref):
    u = (1.0 + eps_ref[0, 0]) * h_ref[...] + p_ref[0, :_N] + p_ref[1, :_N]
    h1 = jnp.dot(u, w1_ref[...], preferred_element_type=jnp.float32) + b1_ref[...]
    h1 = jnp.maximum(h1, 0.0)
    mu = jnp.mean(h1, axis=0, keepdims=True)
    var = jnp.mean((h1 - mu) ** 2, axis=0, keepdims=True)
    h1 = g1_ref[...] * (h1 - mu) * lax.rsqrt(var + 1e-5) + be1_ref[...]
    h2 = jnp.dot(h1, w2_ref[...], preferred_element_type=jnp.float32) + b2_ref[...]
    h2 = jnp.maximum(h2, 0.0)
    mu2 = jnp.mean(h2, axis=0, keepdims=True)
    var2 = jnp.mean((h2 - mu2) ** 2, axis=0, keepdims=True)
    o_ref[...] = g2_ref[...] * (h2 - mu2) * lax.rsqrt(var2 + 1e-5) + be2_ref[...]


def _mlp_tc(h, parts, eps, w1, b1, g1, be1, w2, b2, g2, be2):
    return pl.pallas_call(
        _mlp_body,
        out_shape=jax.ShapeDtypeStruct((_N, _H), jnp.float32),
    )(h, parts, eps.reshape(1, 1), w1, b1.reshape(1, _H), g1.reshape(1, _H),
      be1.reshape(1, _H), w2, b2.reshape(1, _H), g2.reshape(1, _H),
      be2.reshape(1, _H))


def _head_body(h_ref, w1_ref, b1_ref, w2_ref, b2_ref, o_ref):
    z = jnp.dot(h_ref[...], w1_ref[...], preferred_element_type=jnp.float32)
    z = jnp.maximum(z + b1_ref[...], 0.0)
    z = jnp.dot(z, w2_ref[...], preferred_element_type=jnp.float32) + b2_ref[...]
    m = jnp.max(z, axis=-1, keepdims=True)
    e = z - m
    o_ref[...] = e - jnp.log(jnp.sum(jnp.exp(e), axis=-1, keepdims=True))


def _head_tc(h, w1, b1, w2, b2):
    return pl.pallas_call(
        _head_body,
        out_shape=jax.ShapeDtypeStruct((_N, _C), jnp.float32),
    )(h, w1, b1.reshape(1, _H), w2, b2.reshape(1, _C))


def kernel(x, edge_index, params):
    pad_i = jnp.arange(_EPAD - _E, dtype=jnp.int32)
    src = jnp.concatenate([edge_index[0].astype(jnp.int32), pad_i % _N])
    dst = jnp.concatenate([edge_index[1].astype(jnp.int32),
                           _N + pad_i % (_NPAD - _N)])
    src = src.reshape(_NC, _NS, _NG, _G * _CH)
    dst = dst.reshape(_NC, _NS, _NG, _G, _CH)
    h = x
    for l in range(_L):
        pre = "conv%d" % l
        parts = _seg_sum_partials(h, src, dst)
        h = _mlp_tc(
            h, parts, params[pre + "_eps"],
            params[pre + "_W1"], params[pre + "_b1"],
            params[pre + "_g1"], params[pre + "_be1"],
            params[pre + "_W2"], params[pre + "_b2"],
            params[pre + "_g2"], params[pre + "_be2"],
        )
    return _head_tc(h, params["lin1_W"], params["lin1_b"],
                    params["lin2_W"], params["lin2_b"])
